# Initial kernel scaffold; baseline (speedup 1.0000x reference)
import jax
import jax.numpy as jnp
from jax.experimental import pallas as pl


def kernel(x, in_degree, out_degree, atom_W, in_W, out_W, token_W):
    raise NotImplementedError("write your pallas kernel here")


# speedup vs baseline: 2.3985x; 2.3985x over previous
import functools

import jax
import jax.numpy as jnp
from jax import lax
from jax.experimental import pallas as pl
from jax.experimental.pallas import tpu as pltpu
from jax.experimental.pallas import tpu_sc as plsc

B, N, F = 256, 64, 9
H = 768
L = 16
HC = H // L
NW = 32
NB_PER_W = B // NW
C = 8
CHUNKS = NB_PER_W * (N // C)


def _body(x_hbm, ind_hbm, outd_hbm, atom_hbm, in_hbm, out_hbm_w, tok_hbm,
          o_hbm,
          xidx_v, inidx_v, outidx_v, atom_rows, in_rows, out_rows, acc_v,
          tok_v, sem_a, sem_i, sem_o):
    nc = 2
    wid = lax.axis_index("s") * nc + lax.axis_index("c")
    b0 = wid * NB_PER_W
    node0 = b0 * N

    pltpu.sync_copy(x_hbm.at[pl.ds(pl.multiple_of(node0 * F, 8), NB_PER_W * N * F)], xidx_v)
    pltpu.sync_copy(ind_hbm.at[pl.ds(pl.multiple_of(node0, 8), NB_PER_W * N)], inidx_v)
    pltpu.sync_copy(outd_hbm.at[pl.ds(pl.multiple_of(node0, 8), NB_PER_W * N)], outidx_v)
    pltpu.sync_copy(tok_hbm, tok_v)

    def chunk_body(k, carry):
        bi = k // (N // C)
        ci = k % (N // C)
        xoff = pl.multiple_of(k * C * F, 8)
        noff = pl.multiple_of(k * C, 8)
        cp_a = pltpu.async_copy(atom_hbm.at[xidx_v.at[pl.ds(xoff, C * F)]],
                                atom_rows, sem_a)
        cp_i = pltpu.async_copy(in_hbm.at[inidx_v.at[pl.ds(noff, C)]],
                                in_rows, sem_i)
        cp_o = pltpu.async_copy(out_hbm_w.at[outidx_v.at[pl.ds(noff, C)]],
                                out_rows, sem_o)
        cp_a.wait()
        cp_i.wait()
        cp_o.wait()

        def h_body(h, carry2):
            off = pl.ds(h * L, L)
            for c in range(C):
                v = in_rows[c, off] + out_rows[c, off]
                for f in range(F):
                    v = v + atom_rows[c * F + f, off]
                acc_v[pl.ds(c * H + h * L, L)] = v
            return carry2

        lax.fori_loop(0, HC, h_body, 0, unroll=False)

        row0 = (b0 + bi) * (N + 1) + 1 + ci * C
        pltpu.sync_copy(acc_v, o_hbm.at[pl.ds(pl.multiple_of(row0 * H, 8), C * H)])
        return carry

    lax.fori_loop(0, CHUNKS, chunk_body, 0, unroll=False)

    for bi in range(NB_PER_W):
        row0 = (b0 + bi) * (N + 1)
        pltpu.sync_copy(tok_v, o_hbm.at[pl.ds(pl.multiple_of(row0 * H, 8), H)])


@jax.jit
def _graph_node_feature_sc(x_flat, ind_flat, outd_flat, atom_W, in_W, out_W,
                           token_W):
    mesh = plsc.VectorSubcoreMesh(core_axis_name="c", subcore_axis_name="s")
    run = functools.partial(
        pl.kernel,
        mesh=mesh,
        out_type=jax.ShapeDtypeStruct((B * (N + 1) * H,), jnp.float32),
        scratch_types=[
            pltpu.VMEM((NB_PER_W * N * F,), jnp.int32),
            pltpu.VMEM((NB_PER_W * N,), jnp.int32),
            pltpu.VMEM((NB_PER_W * N,), jnp.int32),
            pltpu.VMEM((C * F, H), jnp.float32),
            pltpu.VMEM((C, H), jnp.float32),
            pltpu.VMEM((C, H), jnp.float32),
            pltpu.VMEM((C * H,), jnp.float32),
            pltpu.VMEM((H,), jnp.float32),
            pltpu.SemaphoreType.DMA,
            pltpu.SemaphoreType.DMA,
            pltpu.SemaphoreType.DMA,
        ],
    )(_body)
    return run(x_flat, ind_flat, outd_flat, atom_W, in_W, out_W,
               token_W.reshape(-1))


def kernel(x, in_degree, out_degree, atom_W, in_W, out_W, token_W):
    x_flat = x.reshape(-1).astype(jnp.int32)
    ind_flat = in_degree.reshape(-1).astype(jnp.int32)
    outd_flat = out_degree.reshape(-1).astype(jnp.int32)
    out = _graph_node_feature_sc(x_flat, ind_flat, outd_flat,
                                 atom_W, in_W, out_W, token_W)
    return out.reshape(B, N + 1, H)


# speedup vs baseline: 4.0790x; 1.7006x over previous
import functools

import jax
import jax.numpy as jnp
from jax import lax
from jax.experimental import pallas as pl
from jax.experimental.pallas import tpu as pltpu
from jax.experimental.pallas import tpu_sc as plsc

B, N, F = 256, 64, 9
H = 768
HW = H // 2
L = 16
H2 = H // (2 * L)
NW = 32
NB_PER_W = B // NW
C = 8
CPB = N // C
CHUNKS = NB_PER_W * CPB
TOPBITS = -65536


def _body(x_hbm, ind_hbm, outd_hbm, atom_hbm, in_hbm, outw_hbm, tok_hbm,
          o_hbm,
          xidx_v, inidx_v, outidx_v,
          atom0, atom1, in0, in1, out0, out1, acc0, acc1, tok_v,
          sa0, sa1, si0, si1, so0, so1, sw0, sw1):
    nc = 2
    wid = lax.axis_index("s") * nc + lax.axis_index("c")
    b0 = wid * NB_PER_W
    node0 = b0 * N

    atom_b = (atom0, atom1)
    in_b = (in0, in1)
    out_b = (out0, out1)
    acc_b = (acc0, acc1)
    sa = (sa0, sa1)
    si = (si0, si1)
    so = (so0, so1)
    sw = (sw0, sw1)

    pltpu.sync_copy(x_hbm.at[pl.ds(pl.multiple_of(node0 * F, 8), NB_PER_W * N * F)], xidx_v)
    pltpu.sync_copy(ind_hbm.at[pl.ds(pl.multiple_of(node0, 8), NB_PER_W * N)], inidx_v)
    pltpu.sync_copy(outd_hbm.at[pl.ds(pl.multiple_of(node0, 8), NB_PER_W * N)], outidx_v)
    pltpu.sync_copy(tok_hbm, tok_v)
    for bi in range(NB_PER_W):
        row0 = (b0 + bi) * (N + 1)
        pltpu.sync_copy(tok_v, o_hbm.at[pl.ds(pl.multiple_of(row0 * H, 8), H)])

    iota2 = lax.iota(jnp.int32, L) * 2

    def issue_gathers(k, slot):
        xoff = pl.multiple_of(k * C * F, 8)
        noff = pl.multiple_of(k * C, 8)
        pltpu.async_copy(atom_hbm.at[xidx_v.at[pl.ds(xoff, C * F)]],
                         atom_b[slot], sa[slot])
        pltpu.async_copy(in_hbm.at[inidx_v.at[pl.ds(noff, C)]],
                         in_b[slot], si[slot])
        pltpu.async_copy(outw_hbm.at[outidx_v.at[pl.ds(noff, C)]],
                         out_b[slot], so[slot])

    def wait_gathers(slot):
        pltpu.make_async_copy(atom_hbm.at[xidx_v.at[pl.ds(0, C * F)]],
                              atom_b[slot], sa[slot]).wait()
        pltpu.make_async_copy(in_hbm.at[inidx_v.at[pl.ds(0, C)]],
                              in_b[slot], si[slot]).wait()
        pltpu.make_async_copy(outw_hbm.at[outidx_v.at[pl.ds(0, C)]],
                              out_b[slot], so[slot]).wait()

    def wait_write(slot):
        pltpu.make_async_copy(acc_b[slot], o_hbm.at[pl.ds(0, C * H)],
                              sw[slot]).wait()

    def compute(slot):
        av, iv, ov, accv = atom_b[slot], in_b[slot], out_b[slot], acc_b[slot]

        def widen(v):
            e = plsc.bitcast(lax.shift_left(v, 16), jnp.float32)
            o = plsc.bitcast(lax.bitwise_and(v, TOPBITS), jnp.float32)
            return e, o

        def h_body(h, carry):
            off = pl.ds(h * L, L)
            hbase = h * 2 * L
            for c in range(C):
                e, o = widen(iv[c, off])
                e2, o2 = widen(ov[c, off])
                e, o = e + e2, o + o2
                for f in range(F):
                    ea, oa = widen(av[c * F + f, off])
                    e, o = e + ea, o + oa
                idx_e = iota2 + (c * H + hbase)
                plsc.store_scatter(accv, [idx_e], e)
                plsc.store_scatter(accv, [idx_e + 1], o)
            return carry

        lax.fori_loop(0, H2, h_body, 0, unroll=False)

    def issue_write(k, slot):
        bi = k // CPB
        ci = k % CPB
        row0 = (b0 + bi) * (N + 1) + 1 + ci * C
        pltpu.async_copy(acc_b[slot],
                         o_hbm.at[pl.ds(pl.multiple_of(row0 * H, 8), C * H)],
                         sw[slot])

    issue_gathers(0, 0)

    def pair_body(j, carry):
        k0 = 2 * j
        k1 = k0 + 1
        issue_gathers(k1, 1)

        @pl.when(j > 0)
        def _():
            wait_write(0)

        wait_gathers(0)
        compute(0)
        issue_write(k0, 0)
        issue_gathers(jnp.minimum(k0 + 2, CHUNKS - 1), 0)

        @pl.when(j > 0)
        def _():
            wait_write(1)

        wait_gathers(1)
        compute(1)
        issue_write(k1, 1)
        return carry

    lax.fori_loop(0, CHUNKS // 2, pair_body, 0, unroll=False)

    wait_gathers(0)
    wait_write(0)
    wait_write(1)


@jax.jit
def _graph_node_feature_sc(x_flat, ind_flat, outd_flat, atom_p, in_p, out_p,
                           token_W):
    mesh = plsc.VectorSubcoreMesh(core_axis_name="c", subcore_axis_name="s")
    run = functools.partial(
        pl.kernel,
        mesh=mesh,
        compiler_params=pltpu.CompilerParams(needs_layout_passes=False),
        out_type=jax.ShapeDtypeStruct((B * (N + 1) * H,), jnp.float32),
        scratch_types=[
            pltpu.VMEM((NB_PER_W * N * F,), jnp.int32),
            pltpu.VMEM((NB_PER_W * N,), jnp.int32),
            pltpu.VMEM((NB_PER_W * N,), jnp.int32),
            pltpu.VMEM((C * F, HW), jnp.int32),
            pltpu.VMEM((C * F, HW), jnp.int32),
            pltpu.VMEM((C, HW), jnp.int32),
            pltpu.VMEM((C, HW), jnp.int32),
            pltpu.VMEM((C, HW), jnp.int32),
            pltpu.VMEM((C, HW), jnp.int32),
            pltpu.VMEM((C * H,), jnp.float32),
            pltpu.VMEM((C * H,), jnp.float32),
            pltpu.VMEM((H,), jnp.float32),
        ] + [pltpu.SemaphoreType.DMA] * 8,
    )(_body)
    return run(x_flat, ind_flat, outd_flat, atom_p, in_p, out_p, token_W)


def _pack_bf16(w):
    wb = w.astype(jnp.bfloat16)
    return lax.bitcast_convert_type(
        wb.reshape(w.shape[0], w.shape[1] // 2, 2), jnp.int32)


def kernel(x, in_degree, out_degree, atom_W, in_W, out_W, token_W):
    x_flat = x.reshape(-1).astype(jnp.int32)
    ind_flat = in_degree.reshape(-1).astype(jnp.int32)
    outd_flat = out_degree.reshape(-1).astype(jnp.int32)
    out = _graph_node_feature_sc(x_flat, ind_flat, outd_flat,
                                 _pack_bf16(atom_W), _pack_bf16(in_W),
                                 _pack_bf16(out_W), token_W.reshape(-1))
    return out.reshape(B, N + 1, H)


# speedup vs baseline: 4.1959x; 1.0287x over previous
import functools

import jax
import jax.numpy as jnp
from jax import lax
from jax.experimental import pallas as pl
from jax.experimental.pallas import tpu as pltpu
from jax.experimental.pallas import tpu_sc as plsc

B, N, F = 256, 64, 9
H = 768
HW = H // 2
L = 16
H2 = HW // L
NW = 32
NS = 16
NB_PER_W = B // NW
C = 8
CPB = N // C
CHUNKS = NB_PER_W * CPB
TOPBITS = -65536

NA = 4608 + 1
ND = 512
A_PAD = 4616
IN_BASE = A_PAD
OUT_BASE = A_PAD + ND
SC_ROWS = A_PAD + 2 * ND
A_BLOCKS = (NA - 1) // 8
D_BLOCKS = ND // 8
BLOCKS = A_BLOCKS + 2 * D_BLOCKS
BLK_PER_TILE = BLOCKS // NS


def _bf16_round(u):
    return u + 32767 + lax.bitwise_and(lax.shift_right_logical(u, 16), 1)


def _pack2(lo, hi):
    rl = _bf16_round(plsc.bitcast(lo, jnp.int32))
    rh = _bf16_round(plsc.bitcast(hi, jnp.int32))
    return lax.bitwise_or(lax.shift_right_logical(rl, 16),
                          lax.bitwise_and(rh, TOPBITS))


def _body(x_hbm, ind_hbm, outd_hbm, atom_hbm, in_hbm, outw_hbm, tok_hbm,
          o_hbm, packed_hbm,
          xidx_v, inidx_v, outidx_v,
          atom0, atom1, in0, in1, out0, out1, acc0, acc1, tok_v,
          fbuf, pbuf,
          sa0, sa1, si0, si1, so0, so1, sw0, sw1):
    scid = lax.axis_index("c")
    tid = lax.axis_index("s")
    wid = tid * 2 + scid
    b0 = wid * NB_PER_W
    node0 = b0 * N
    dst_base = scid * SC_ROWS

    atom_b = (atom0, atom1)
    in_b = (in0, in1)
    out_b = (out0, out1)
    acc_b = (acc0, acc1)
    sa = (sa0, sa1)
    si = (si0, si1)
    so = (so0, so1)
    sw = (sw0, sw1)

    pltpu.sync_copy(x_hbm.at[pl.ds(pl.multiple_of(node0 * F, 8), NB_PER_W * N * F)], xidx_v)
    pltpu.sync_copy(ind_hbm.at[pl.ds(pl.multiple_of(node0, 8), NB_PER_W * N)], inidx_v)
    pltpu.sync_copy(outd_hbm.at[pl.ds(pl.multiple_of(node0, 8), NB_PER_W * N)], outidx_v)
    pltpu.sync_copy(tok_hbm, tok_v)
    for bi in range(NB_PER_W):
        row0 = (b0 + bi) * (N + 1)
        pltpu.sync_copy(tok_v, o_hbm.at[pl.ds(pl.multiple_of(row0 * H, 8), H)])

    def pack_rows(nrows):
        def g_body(g, cc):
            lo_off = pl.ds(g * L, L)
            hi_off = pl.ds(HW + g * L, L)
            for r in range(nrows):
                pbuf[r, lo_off] = _pack2(fbuf[r, lo_off], fbuf[r, hi_off])
            return cc
        lax.fori_loop(0, H2, g_body, 0, unroll=False)

    def blk_body(j, cc):
        b = j * NS + tid

        @pl.when(b < A_BLOCKS)
        def _():
            r0 = pl.multiple_of(b * 8, 8)
            pltpu.sync_copy(atom_hbm.at[pl.ds(r0, 8)], fbuf)
            pack_rows(8)
            pltpu.sync_copy(pbuf, packed_hbm.at[pl.ds(pl.multiple_of(dst_base + r0, 8), 8)])

        @pl.when((b >= A_BLOCKS) & (b < A_BLOCKS + D_BLOCKS))
        def _():
            r0 = pl.multiple_of((b - A_BLOCKS) * 8, 8)
            pltpu.sync_copy(in_hbm.at[pl.ds(r0, 8)], fbuf)
            pack_rows(8)
            pltpu.sync_copy(pbuf, packed_hbm.at[pl.ds(pl.multiple_of(dst_base + IN_BASE + r0, 8), 8)])

        @pl.when(b >= A_BLOCKS + D_BLOCKS)
        def _():
            r0 = pl.multiple_of((b - A_BLOCKS - D_BLOCKS) * 8, 8)
            pltpu.sync_copy(outw_hbm.at[pl.ds(r0, 8)], fbuf)
            pack_rows(8)
            pltpu.sync_copy(pbuf, packed_hbm.at[pl.ds(pl.multiple_of(dst_base + OUT_BASE + r0, 8), 8)])

        return cc

    lax.fori_loop(0, BLK_PER_TILE, blk_body, 0, unroll=False)

    @pl.when(tid == 0)
    def _():
        pltpu.sync_copy(atom_hbm.at[pl.ds(NA - 1, 1)], fbuf.at[pl.ds(0, 1)])
        pack_rows(1)
        pltpu.sync_copy(pbuf.at[pl.ds(0, 1)],
                        packed_hbm.at[pl.ds(pl.multiple_of(dst_base + NA - 1, 8), 1)])

    def xadj_body(i, cc):
        off = pl.ds(i * L, L)
        xidx_v[off] = xidx_v[off] + dst_base
        return cc

    lax.fori_loop(0, NB_PER_W * N * F // L, xadj_body, 0, unroll=False)

    def dadj_body(i, cc):
        off = pl.ds(i * L, L)
        inidx_v[off] = inidx_v[off] + (dst_base + IN_BASE)
        outidx_v[off] = outidx_v[off] + (dst_base + OUT_BASE)
        return cc

    lax.fori_loop(0, NB_PER_W * N // L, dadj_body, 0, unroll=False)

    plsc.subcore_barrier()

    def issue_gathers(k, slot):
        xoff = pl.multiple_of(k * C * F, 8)
        noff = pl.multiple_of(k * C, 8)
        pltpu.async_copy(packed_hbm.at[xidx_v.at[pl.ds(xoff, C * F)]],
                         atom_b[slot], sa[slot])
        pltpu.async_copy(packed_hbm.at[inidx_v.at[pl.ds(noff, C)]],
                         in_b[slot], si[slot])
        pltpu.async_copy(packed_hbm.at[outidx_v.at[pl.ds(noff, C)]],
                         out_b[slot], so[slot])

    def wait_gathers(slot):
        pltpu.make_async_copy(packed_hbm.at[xidx_v.at[pl.ds(0, C * F)]],
                              atom_b[slot], sa[slot]).wait()
        pltpu.make_async_copy(packed_hbm.at[inidx_v.at[pl.ds(0, C)]],
                              in_b[slot], si[slot]).wait()
        pltpu.make_async_copy(packed_hbm.at[outidx_v.at[pl.ds(0, C)]],
                              out_b[slot], so[slot]).wait()

    def wait_write(slot):
        pltpu.make_async_copy(acc_b[slot], o_hbm.at[pl.ds(0, C * H)],
                              sw[slot]).wait()

    def compute(slot):
        av, iv, ov, accv = atom_b[slot], in_b[slot], out_b[slot], acc_b[slot]

        def widen(v):
            e = plsc.bitcast(lax.shift_left(v, 16), jnp.float32)
            o = plsc.bitcast(lax.bitwise_and(v, TOPBITS), jnp.float32)
            return e, o

        def h_body(g, carry):
            off = pl.ds(g * L, L)
            gbase = g * L
            for c in range(C):
                e, o = widen(iv[c, off])
                e2, o2 = widen(ov[c, off])
                e, o = e + e2, o + o2
                for f in range(F):
                    ea, oa = widen(av[c * F + f, off])
                    e, o = e + ea, o + oa
                accv[pl.ds(c * H + gbase, L)] = e
                accv[pl.ds(c * H + HW + gbase, L)] = o
            return carry

        lax.fori_loop(0, H2, h_body, 0, unroll=False)

    def issue_write(k, slot):
        bi = k // CPB
        ci = k % CPB
        row0 = (b0 + bi) * (N + 1) + 1 + ci * C
        pltpu.async_copy(acc_b[slot],
                         o_hbm.at[pl.ds(pl.multiple_of(row0 * H, 8), C * H)],
                         sw[slot])

    issue_gathers(0, 0)

    def pair_body(j, carry):
        k0 = 2 * j
        k1 = k0 + 1
        issue_gathers(k1, 1)

        @pl.when(j > 0)
        def _():
            wait_write(0)

        wait_gathers(0)
        compute(0)
        issue_write(k0, 0)
        issue_gathers(jnp.minimum(k0 + 2, CHUNKS - 1), 0)

        @pl.when(j > 0)
        def _():
            wait_write(1)

        wait_gathers(1)
        compute(1)
        issue_write(k1, 1)
        return carry

    lax.fori_loop(0, CHUNKS // 2, pair_body, 0, unroll=False)

    wait_gathers(0)
    wait_write(0)
    wait_write(1)


@jax.jit
def _graph_node_feature_sc(x_flat, ind_flat, outd_flat, atom_W, in_W, out_W,
                           token_W):
    mesh = plsc.VectorSubcoreMesh(core_axis_name="c", subcore_axis_name="s")
    run = functools.partial(
        pl.kernel,
        mesh=mesh,
        compiler_params=pltpu.CompilerParams(needs_layout_passes=False),
        out_type=[
            jax.ShapeDtypeStruct((B * (N + 1) * H,), jnp.float32),
            jax.ShapeDtypeStruct((2 * SC_ROWS, HW), jnp.int32),
        ],
        scratch_types=[
            pltpu.VMEM((NB_PER_W * N * F,), jnp.int32),
            pltpu.VMEM((NB_PER_W * N,), jnp.int32),
            pltpu.VMEM((NB_PER_W * N,), jnp.int32),
            pltpu.VMEM((C * F, HW), jnp.int32),
            pltpu.VMEM((C * F, HW), jnp.int32),
            pltpu.VMEM((C, HW), jnp.int32),
            pltpu.VMEM((C, HW), jnp.int32),
            pltpu.VMEM((C, HW), jnp.int32),
            pltpu.VMEM((C, HW), jnp.int32),
            pltpu.VMEM((C * H,), jnp.float32),
            pltpu.VMEM((C * H,), jnp.float32),
            pltpu.VMEM((H,), jnp.float32),
            pltpu.VMEM((8, H), jnp.float32),
            pltpu.VMEM((8, HW), jnp.int32),
        ] + [pltpu.SemaphoreType.DMA] * 8,
    )(_body)
    out, _ = run(x_flat, ind_flat, outd_flat, atom_W, in_W, out_W, token_W)
    return out


def kernel(x, in_degree, out_degree, atom_W, in_W, out_W, token_W):
    x_flat = x.reshape(-1).astype(jnp.int32)
    ind_flat = in_degree.reshape(-1).astype(jnp.int32)
    outd_flat = out_degree.reshape(-1).astype(jnp.int32)
    out = _graph_node_feature_sc(x_flat, ind_flat, outd_flat,
                                 atom_W, in_W, out_W, token_W.reshape(-1))
    return out.reshape(B, N + 1, H)


# speedup vs baseline: 4.7056x; 1.1215x over previous
import functools

import jax
import jax.numpy as jnp
from jax import lax
from jax.experimental import pallas as pl
from jax.experimental.pallas import tpu as pltpu
from jax.experimental.pallas import tpu_sc as plsc

B, N, F = 256, 64, 9
H = 768
HW = H // 2
L = 16
H2 = HW // L
NW = 32
NS = 16
NB_PER_W = B // NW
NODES_W = NB_PER_W * N
C = 8
CPB = N // C
CHUNKS = NB_PER_W * CPB
TOPBITS = -65536

NA = 4608 + 1
ND = 512
A_PAD = 4616
IN_BASE = A_PAD
OUT_BASE = A_PAD + ND
SC_ROWS = A_PAD + 2 * ND
A_BLOCKS = (NA - 1) // 8
D_BLOCKS = ND // 8
BLOCKS = A_BLOCKS + 2 * D_BLOCKS
BLK_PER_TILE = BLOCKS // NS


def _bf16_round(u):
    return u + 32767 + lax.bitwise_and(lax.shift_right_logical(u, 16), 1)


def _pack2(lo, hi):
    rl = _bf16_round(plsc.bitcast(lo, jnp.int32))
    rh = _bf16_round(plsc.bitcast(hi, jnp.int32))
    return lax.bitwise_or(lax.shift_right_logical(rl, 16),
                          lax.bitwise_and(rh, TOPBITS))


def _body(x_hbm, ind_hbm, outd_hbm, atom_hbm, in_hbm, outw_hbm, tok_hbm,
          o_hbm, packed_hbm,
          xstage, instage, outstage,
          xidx_v, inidx_v, outidx_v,
          atom0, atom1, in0, in1, out0, out1, acc0, acc1, tok_v,
          fbuf0, fbuf1, pbuf0, pbuf1,
          sa0, sa1, si0, si1, so0, so1, sw0, sw1, sr0, sr1, sp0, sp1):
    scid = lax.axis_index("c")
    tid = lax.axis_index("s")
    wid = tid * 2 + scid
    b0 = wid * NB_PER_W
    node0 = b0 * N
    dst_base = scid * SC_ROWS

    atom_b = (atom0, atom1)
    in_b = (in0, in1)
    out_b = (out0, out1)
    acc_b = (acc0, acc1)
    fbuf = (fbuf0, fbuf1)
    pbuf = (pbuf0, pbuf1)
    sa = (sa0, sa1)
    si = (si0, si1)
    so = (so0, so1)
    sw = (sw0, sw1)
    sr = (sr0, sr1)
    sp = (sp0, sp1)

    pltpu.sync_copy(ind_hbm.at[pl.ds(pl.multiple_of(b0, 8), NB_PER_W)], instage)
    pltpu.sync_copy(outd_hbm.at[pl.ds(pl.multiple_of(b0, 8), NB_PER_W)], outstage)
    pltpu.sync_copy(tok_hbm, tok_v)
    for bi in range(NB_PER_W):
        row0 = (b0 + bi) * (N + 1)
        pltpu.sync_copy(tok_v, o_hbm.at[pl.ds(pl.multiple_of(row0 * H, 8), H)])

    def pack_rows(fb, pb, nrows):
        def g_body(g, cc):
            lo_off = pl.ds(g * L, L)
            hi_off = pl.ds(HW + g * L, L)
            for r in range(nrows):
                pb[r, lo_off] = _pack2(fb[r, lo_off], fb[r, hi_off])
            return cc
        lax.fori_loop(0, H2, g_body, 0, unroll=False)

    def p0_issue_read(bl, slot):
        b = bl * NS + tid

        @pl.when(b < A_BLOCKS)
        def _():
            r0 = pl.multiple_of(b * 8, 8)
            pltpu.async_copy(atom_hbm.at[pl.ds(r0, 8)], fbuf[slot], sr[slot])

        @pl.when((b >= A_BLOCKS) & (b < A_BLOCKS + D_BLOCKS))
        def _():
            r0 = pl.multiple_of((b - A_BLOCKS) * 8, 8)
            pltpu.async_copy(in_hbm.at[pl.ds(r0, 8)], fbuf[slot], sr[slot])

        @pl.when(b >= A_BLOCKS + D_BLOCKS)
        def _():
            r0 = pl.multiple_of((b - A_BLOCKS - D_BLOCKS) * 8, 8)
            pltpu.async_copy(outw_hbm.at[pl.ds(r0, 8)], fbuf[slot], sr[slot])

    def p0_wait_read(slot):
        pltpu.make_async_copy(atom_hbm.at[pl.ds(0, 8)], fbuf[slot],
                              sr[slot]).wait()

    def p0_issue_write(bl, slot):
        b = bl * NS + tid
        dst = pl.multiple_of(
            dst_base + b * 8 + jnp.where(b >= A_BLOCKS, 8, 0), 8)
        pltpu.async_copy(pbuf[slot], packed_hbm.at[pl.ds(dst, 8)], sp[slot])

    def p0_wait_write(slot):
        pltpu.make_async_copy(pbuf[slot], packed_hbm.at[pl.ds(0, 8)],
                              sp[slot]).wait()

    p0_issue_read(0, 0)

    def p0_pair(jj, cc):
        p0_issue_read(2 * jj + 1, 1)
        p0_wait_read(0)

        @pl.when(jj > 0)
        def _():
            p0_wait_write(0)

        pack_rows(fbuf[0], pbuf[0], 8)
        p0_issue_write(2 * jj, 0)
        p0_issue_read(jnp.minimum(2 * jj + 2, BLK_PER_TILE - 1), 0)
        p0_wait_read(1)

        @pl.when(jj > 0)
        def _():
            p0_wait_write(1)

        pack_rows(fbuf[1], pbuf[1], 8)
        p0_issue_write(2 * jj + 1, 1)
        return cc

    lax.fori_loop(0, BLK_PER_TILE // 2, p0_pair, 0, unroll=False)
    p0_wait_read(0)
    p0_wait_write(0)
    p0_wait_write(1)

    @pl.when(tid == 0)
    def _():
        pltpu.sync_copy(atom_hbm.at[pl.ds(NA - 1, 1)], fbuf0.at[pl.ds(0, 1)])
        pack_rows(fbuf0, pbuf0, 1)
        pltpu.sync_copy(pbuf0.at[pl.ds(0, 1)],
                        packed_hbm.at[pl.ds(pl.multiple_of(dst_base + NA - 1, 8), 1)])

    iota = lax.iota(jnp.int32, L)
    XS = NODES_W // 4

    for s in range(4):
        pltpu.sync_copy(x_hbm.at[pl.ds(pl.multiple_of(node0 + s * XS, 8), XS)],
                        xstage)

        def xfl_body(i, cc):
            flat = i * L + iota
            v = plsc.load_gather(xstage, [lax.div(flat, F), lax.rem(flat, F)])
            xidx_v[pl.ds((s * XS * F // L + i) * L, L)] = v + dst_base
            return cc

        lax.fori_loop(0, XS * F // L, xfl_body, 0, unroll=False)

    for bi in range(NB_PER_W):
        for g in range(N // L):
            off = pl.ds(g * L, L)
            dst = pl.ds(bi * N + g * L, L)
            inidx_v[dst] = instage[bi, off] + (dst_base + IN_BASE)
            outidx_v[dst] = outstage[bi, off] + (dst_base + OUT_BASE)

    plsc.subcore_barrier()

    def issue_gathers(k, slot):
        xoff = pl.multiple_of(k * C * F, 8)
        noff = pl.multiple_of(k * C, 8)
        pltpu.async_copy(packed_hbm.at[xidx_v.at[pl.ds(xoff, C * F)]],
                         atom_b[slot], sa[slot])
        pltpu.async_copy(packed_hbm.at[inidx_v.at[pl.ds(noff, C)]],
                         in_b[slot], si[slot])
        pltpu.async_copy(packed_hbm.at[outidx_v.at[pl.ds(noff, C)]],
                         out_b[slot], so[slot])

    def wait_gathers(slot):
        pltpu.make_async_copy(packed_hbm.at[xidx_v.at[pl.ds(0, C * F)]],
                              atom_b[slot], sa[slot]).wait()
        pltpu.make_async_copy(packed_hbm.at[inidx_v.at[pl.ds(0, C)]],
                              in_b[slot], si[slot]).wait()
        pltpu.make_async_copy(packed_hbm.at[outidx_v.at[pl.ds(0, C)]],
                              out_b[slot], so[slot]).wait()

    def wait_write(slot):
        pltpu.make_async_copy(acc_b[slot], o_hbm.at[pl.ds(0, C * H)],
                              sw[slot]).wait()

    def compute(slot):
        av, iv, ov, accv = atom_b[slot], in_b[slot], out_b[slot], acc_b[slot]

        def widen(v):
            e = plsc.bitcast(lax.shift_left(v, 16), jnp.float32)
            o = plsc.bitcast(lax.bitwise_and(v, TOPBITS), jnp.float32)
            return e, o

        def h_body(g, carry):
            off = pl.ds(g * L, L)
            gbase = g * L
            for c in range(C):
                e, o = widen(iv[c, off])
                e2, o2 = widen(ov[c, off])
                e, o = e + e2, o + o2
                for f in range(F):
                    ea, oa = widen(av[c * F + f, off])
                    e, o = e + ea, o + oa
                accv[pl.ds(c * H + gbase, L)] = e
                accv[pl.ds(c * H + HW + gbase, L)] = o
            return carry

        lax.fori_loop(0, H2, h_body, 0, unroll=False)

    def issue_write(k, slot):
        bi = k // CPB
        ci = k % CPB
        row0 = (b0 + bi) * (N + 1) + 1 + ci * C
        pltpu.async_copy(acc_b[slot],
                         o_hbm.at[pl.ds(pl.multiple_of(row0 * H, 8), C * H)],
                         sw[slot])

    issue_gathers(0, 0)

    def pair_body(j, carry):
        k0 = 2 * j
        k1 = k0 + 1
        issue_gathers(k1, 1)

        @pl.when(j > 0)
        def _():
            wait_write(0)

        wait_gathers(0)
        compute(0)
        issue_write(k0, 0)
        issue_gathers(jnp.minimum(k0 + 2, CHUNKS - 1), 0)

        @pl.when(j > 0)
        def _():
            wait_write(1)

        wait_gathers(1)
        compute(1)
        issue_write(k1, 1)
        return carry

    lax.fori_loop(0, CHUNKS // 2, pair_body, 0, unroll=False)

    wait_gathers(0)
    wait_write(0)
    wait_write(1)


@jax.jit
def _graph_node_feature_sc(x2d, ind2d, outd2d, atom_W, in_W, out_W,
                           token_W):
    mesh = plsc.VectorSubcoreMesh(core_axis_name="c", subcore_axis_name="s")
    run = functools.partial(
        pl.kernel,
        mesh=mesh,
        compiler_params=pltpu.CompilerParams(needs_layout_passes=False),
        out_type=[
            jax.ShapeDtypeStruct((B * (N + 1) * H,), jnp.float32),
            jax.ShapeDtypeStruct((2 * SC_ROWS, HW), jnp.int32),
        ],
        scratch_types=[
            pltpu.VMEM((NODES_W // 4, F), jnp.int32),
            pltpu.VMEM((NB_PER_W, N), jnp.int32),
            pltpu.VMEM((NB_PER_W, N), jnp.int32),
            pltpu.VMEM((NODES_W * F,), jnp.int32),
            pltpu.VMEM((NODES_W,), jnp.int32),
            pltpu.VMEM((NODES_W,), jnp.int32),
            pltpu.VMEM((C * F, HW), jnp.int32),
            pltpu.VMEM((C * F, HW), jnp.int32),
            pltpu.VMEM((C, HW), jnp.int32),
            pltpu.VMEM((C, HW), jnp.int32),
            pltpu.VMEM((C, HW), jnp.int32),
            pltpu.VMEM((C, HW), jnp.int32),
            pltpu.VMEM((C * H,), jnp.float32),
            pltpu.VMEM((C * H,), jnp.float32),
            pltpu.VMEM((H,), jnp.float32),
            pltpu.VMEM((8, H), jnp.float32),
            pltpu.VMEM((8, H), jnp.float32),
            pltpu.VMEM((8, HW), jnp.int32),
            pltpu.VMEM((8, HW), jnp.int32),
        ] + [pltpu.SemaphoreType.DMA] * 12,
    )(_body)
    out, _ = run(x2d, ind2d, outd2d, atom_W, in_W, out_W, token_W)
    return out


def kernel(x, in_degree, out_degree, atom_W, in_W, out_W, token_W):
    out = _graph_node_feature_sc(x.reshape(B * N, F).astype(jnp.int32),
                                 in_degree.astype(jnp.int32),
                                 out_degree.astype(jnp.int32),
                                 atom_W, in_W, out_W, token_W.reshape(-1))
    return out.reshape(B, N + 1, H)
